# baseline (speedup 1.0000x reference)
import functools

import jax
import jax.numpy as jnp
from jax import lax
from jax.experimental import pallas as pl
from jax.experimental.pallas import tpu as pltpu
from jax.experimental.pallas import tpu_sc as plsc

NC = 2
NS = 16
LANE = 128


def _ln_relu(h, g, be):
    mu = jnp.mean(h, axis=-1, keepdims=True)
    d = h - mu
    var = jnp.mean(d * d, axis=-1, keepdims=True)
    inv = lax.rsqrt(var + 1e-5)
    return jnp.maximum(d * inv * g + be, 0.0)


def _pre_body(x_ref, wd_ref, ws_ref, b_ref, p_ref, q_ref):
    x = x_ref[...]
    p_ref[...] = jnp.dot(x, wd_ref[...], preferred_element_type=jnp.float32) + b_ref[...]
    q_ref[...] = jnp.dot(x, ws_ref[...], preferred_element_type=jnp.float32)


def _mlp_body(h_ref, g1_ref, be1_ref, w2_ref, b2_ref, g2_ref, be2_ref, o_ref):
    h = _ln_relu(h_ref[...], g1_ref[...], be1_ref[...])
    h = jnp.dot(h, w2_ref[...], preferred_element_type=jnp.float32) + b2_ref[...]
    o_ref[...] = _ln_relu(h, g2_ref[...], be2_ref[...])


def _upd_body(x_ref, parts_ref, wx_ref, wa_ref, b_ref, g_ref, be_ref, o_ref):
    x = x_ref[...]
    agg = parts_ref[0] + parts_ref[1]
    h = (jnp.dot(x, wx_ref[...], preferred_element_type=jnp.float32)
         + jnp.dot(agg, wa_ref[...], preferred_element_type=jnp.float32)
         + b_ref[...])
    o_ref[...] = x + _ln_relu(h, g_ref[...], be_ref[...])


def _full(shape):
    return pl.BlockSpec(shape, lambda i: (0,) * len(shape))


def _tc_pre(x_pad, wd, ws, b, n_pad, d, bn):
    grid = (n_pad // bn,)
    return pl.pallas_call(
        _pre_body,
        grid=grid,
        in_specs=[pl.BlockSpec((bn, d), lambda i: (i, 0)),
                  _full((d, d)), _full((d, d)), _full((1, d))],
        out_specs=[pl.BlockSpec((bn, d), lambda i: (i, 0)),
                   pl.BlockSpec((bn, d), lambda i: (i, 0))],
        out_shape=[jax.ShapeDtypeStruct((n_pad, d), jnp.float32),
                   jax.ShapeDtypeStruct((n_pad, d), jnp.float32)],
    )(x_pad, wd, ws, b)


def _tc_mlp(msg, g1, be1, w2, b2, g2, be2, e_pad, d, be_blk):
    grid = (e_pad // be_blk,)
    return pl.pallas_call(
        _mlp_body,
        grid=grid,
        in_specs=[pl.BlockSpec((be_blk, d), lambda i: (i, 0)),
                  _full((1, d)), _full((1, d)), _full((d, d)),
                  _full((1, d)), _full((1, d)), _full((1, d))],
        out_specs=pl.BlockSpec((be_blk, d), lambda i: (i, 0)),
        out_shape=jax.ShapeDtypeStruct((e_pad, d), jnp.float32),
    )(msg, g1, be1, w2, b2, g2, be2)


def _tc_update(x, parts, wx, wa, b, g, be, n, d, bn):
    grid = (n // bn,)
    return pl.pallas_call(
        _upd_body,
        grid=grid,
        in_specs=[pl.BlockSpec((bn, d), lambda i: (i, 0)),
                  pl.BlockSpec((2, bn, d), lambda i: (0, i, 0)),
                  _full((d, d)), _full((d, d)),
                  _full((1, d)), _full((1, d)), _full((1, d))],
        out_specs=pl.BlockSpec((bn, d), lambda i: (i, 0)),
        out_shape=jax.ShapeDtypeStruct((n, d), jnp.float32),
    )(x, parts, wx, wa, b, g, be)


def _make_gather(k0, k1, e_pad, n_pad, d):
    mesh = plsc.VectorSubcoreMesh(core_axis_name="c", subcore_axis_name="s")
    assert k0 >= 2 and k0 % 2 == 1 and k1 >= 2 and k1 % 2 == 1
    kmax = max(k0, k1)

    @functools.partial(
        pl.kernel,
        out_type=jax.ShapeDtypeStruct((e_pad, d), jnp.float32),
        mesh=mesh,
        scratch_types=[
            pltpu.VMEM((kmax, LANE), jnp.int32),
            pltpu.VMEM((kmax, LANE), jnp.int32),
            pltpu.VMEM((LANE, d), jnp.float32),
            pltpu.VMEM((LANE, d), jnp.float32),
            pltpu.VMEM((LANE, d), jnp.float32),
            pltpu.VMEM((LANE, d), jnp.float32),
            pltpu.SemaphoreType.DMA, pltpu.SemaphoreType.DMA,
            pltpu.SemaphoreType.DMA, pltpu.SemaphoreType.DMA,
            pltpu.SemaphoreType.DMA, pltpu.SemaphoreType.DMA,
        ],
    )
    def gather_kernel(p_hbm, q_hbm, dst_hbm, src_hbm, msg_hbm,
                      idxd, idxs, bp0, bp1, bq0, bq1,
                      sp0, sp1, sq0, sq1, so0, so1):
        c = lax.axis_index("c")
        s = lax.axis_index("s")
        w = c * NS + s
        kc = jnp.where(c == 0, k0, k1)
        row0 = jnp.where(c == 0, s * k0, NS * k0 + s * k1)
        pltpu.sync_copy(dst_hbm.at[w], idxd)
        pltpu.sync_copy(src_hbm.at[w], idxs)
        bp = (bp0, bp1)
        bq = (bq0, bq1)
        sp = (sp0, sp1)
        sq = (sq0, sq1)
        so = (so0, so1)

        def issue_gathers(j, b):
            pltpu.async_copy(p_hbm.at[idxd.at[j]], bp[b], sp[b])
            pltpu.async_copy(q_hbm.at[idxs.at[j]], bq[b], sq[b])

        def wait_gathers(j, b):
            pltpu.make_async_copy(p_hbm.at[idxd.at[j]], bp[b], sp[b]).wait()
            pltpu.make_async_copy(q_hbm.at[idxs.at[j]], bq[b], sq[b]).wait()

        def add_chunk(b):
            def _add(r, carry):
                for rr in range(4):
                    for t in range(d // 16):
                        ix = (r * 4 + rr, pl.ds(t * 16, 16))
                        plsc.addupdate(bp[b].at[ix], bq[b][ix])
                return carry

            lax.fori_loop(0, LANE // 4, _add, 0)

        def store(j, b):
            pltpu.async_copy(bp[b], msg_hbm.at[pl.ds((row0 + j) * LANE, LANE)],
                             so[b])

        def wait_store(j, b):
            pltpu.make_async_copy(
                bp[b], msg_hbm.at[pl.ds((row0 + j) * LANE, LANE)], so[b]).wait()

        issue_gathers(0, 0)
        wait_gathers(0, 0)
        issue_gathers(1, 1)
        add_chunk(0)
        store(0, 0)

        def outer(i, carry):
            for off in (1, 2):
                j = 2 * i + off
                b = off % 2
                wait_gathers(j, b)
                wait_store(j - 1, 1 - b)

                @pl.when(j < kc - 1)
                def _issue():
                    issue_gathers(j + 1, 1 - b)

                add_chunk(b)
                store(j, b)
            return carry

        lax.fori_loop(0, (kc - 1) // 2, outer, 0)
        wait_store(kc - 1, 0)

    return gather_kernel


def _make_scatter(k_chunks, rows2d, n_pad, d):
    mesh = plsc.VectorSubcoreMesh(core_axis_name="c", subcore_axis_name="s")
    zrows = n_pad // NS // LANE

    assert k_chunks >= 2 and k_chunks % 2 == 1

    @functools.partial(
        pl.kernel,
        out_type=jax.ShapeDtypeStruct((NC, n_pad, d), jnp.float32),
        mesh=mesh,
        scratch_types=[
            pltpu.VMEM((k_chunks, LANE), jnp.int32),
            pltpu.VMEM((LANE, d), jnp.float32),
            pltpu.VMEM((LANE, d), jnp.float32),
            pltpu.VMEM_SHARED((n_pad, d), jnp.float32),
            pltpu.SemaphoreType.DMA, pltpu.SemaphoreType.DMA,
            pltpu.SemaphoreType.DMA, pltpu.SemaphoreType.DMA,
        ],
    )
    def scatter_kernel(m_hbm, dst_hbm, out_hbm, idxd, buf0, buf1, agg,
                       sl0, sl1, ss0, ss1):
        c = lax.axis_index("c")
        s = lax.axis_index("s")
        buf = (buf0, buf1)
        sl = (sl0, sl1)
        ss = (ss0, ss1)

        zero16 = jnp.zeros((16,), jnp.float32)

        def zrow(r, carry):
            for t in range(d // 16):
                buf0[r, pl.ds(t * 16, 16)] = zero16
            return carry

        lax.fori_loop(0, LANE, zrow, 0)
        for z in range(zrows):
            pltpu.sync_copy(buf0, agg.at[pl.ds((s * zrows + z) * LANE, LANE)])
        plsc.subcore_barrier()

        w = c * NS + s
        row0 = w * k_chunks
        pltpu.sync_copy(dst_hbm.at[w], idxd)

        def issue_load(j, b):
            pltpu.async_copy(m_hbm.at[pl.ds((row0 + j) * LANE, LANE)], buf[b],
                             sl[b])

        def wait_load(j, b):
            pltpu.make_async_copy(
                m_hbm.at[pl.ds((row0 + j) * LANE, LANE)], buf[b], sl[b]).wait()

        def issue_scat(j, b):
            pltpu.async_copy(buf[b], agg.at[idxd.at[j]], ss[b], add=True)

        def wait_scat(j, b):
            pltpu.make_async_copy(buf[b], agg.at[idxd.at[j]], ss[b]).wait()

        issue_load(0, 0)
        wait_load(0, 0)
        issue_load(1, 1)
        issue_scat(0, 0)

        def chunk(i, carry):
            for off in (1, 2):
                j = 2 * i + off
                b = off % 2
                wait_load(j, b)
                wait_scat(j - 1, 1 - b)

                @pl.when(j < k_chunks - 1)
                def _issue():
                    issue_load(j + 1, 1 - b)

                issue_scat(j, b)
            return carry

        lax.fori_loop(0, (k_chunks - 1) // 2, chunk, 0)
        wait_scat(k_chunks - 1, (k_chunks - 1) % 2)
        plsc.subcore_barrier()

        rpt = n_pad // NS
        pltpu.sync_copy(agg.at[pl.ds(s * rpt, rpt)],
                        out_hbm.at[c, pl.ds(s * rpt, rpt)])

    return scatter_kernel


def kernel(node_features, edge_index, params):
    n, d = node_features.shape
    e = edge_index.shape[1]
    nw = NC * NS

    k_chunks = -(-e // (LANE * nw))
    e_pad = k_chunks * LANE * nw
    rows2d = e_pad // LANE
    n_pad = -(-(n + 1) // (NS * LANE)) * (NS * LANE)

    k0 = (rows2d // NS * 16 // 25) | 1
    k1 = rows2d // NS - k0
    kmax = max(k0, k1)

    src = edge_index[0]
    dst = edge_index[1]
    pad_idx = jnp.full((e_pad - e,), n, dtype=jnp.int32)
    src2 = jnp.concatenate([src, pad_idx]).reshape(nw, k_chunks, LANE)
    dst2 = jnp.concatenate([dst, pad_idx]).reshape(nw, k_chunks, LANE)

    w_ids = jnp.arange(nw)
    c_ids, s_ids = w_ids // NS, w_ids % NS
    starts = jnp.where(c_ids == 0, s_ids * k0, NS * k0 + s_ids * k1)
    flat_len = rows2d + kmax
    offs = (starts[:, None, None] * LANE
            + jnp.arange(kmax * LANE).reshape(kmax, LANE))
    padg = jnp.full((flat_len * LANE - e,), n, dtype=jnp.int32)
    srcg = jnp.concatenate([src, padg])[offs]
    dstg = jnp.concatenate([dst, padg])[offs]

    gather_k = _make_gather(k0, k1, e_pad, n_pad, d)
    scatter_k = _make_scatter(k_chunks, rows2d, n_pad, d)

    bn_pre = 1024 if n_pad % 1024 == 0 else NS * LANE
    bn_upd = 1000 if n % 1000 == 0 else n
    be_blk = 2048 if e_pad % 2048 == 0 else LANE * nw

    x = node_features
    for bp in params:
        l1, l2 = bp["msg"]
        l3 = bp["upd"][0]
        x_pad = jnp.pad(x, ((0, n_pad - n), (0, 0)))
        p, q = _tc_pre(x_pad, l1["W"][:d], l1["W"][d:], l1["b"][None],
                       n_pad, d, bn_pre)
        msg = gather_k(p, q, dstg, srcg)
        m = _tc_mlp(msg, l1["g"][None], l1["be"][None], l2["W"], l2["b"][None],
                    l2["g"][None], l2["be"][None], e_pad, d, be_blk)
        parts = scatter_k(m, dst2)
        x = _tc_update(x, parts, l3["W"][:d], l3["W"][d:], l3["b"][None],
                       l3["g"][None], l3["be"][None], n, d, bn_upd)
    return x


# speedup vs baseline: 1.1012x; 1.1012x over previous
import functools

import jax
import jax.numpy as jnp
from jax import lax
from jax.experimental import pallas as pl
from jax.experimental.pallas import tpu as pltpu
from jax.experimental.pallas import tpu_sc as plsc

NC = 2
NS = 16
LANE = 128


def _ln_relu(h, g, be):
    mu = jnp.mean(h, axis=-1, keepdims=True)
    d = h - mu
    var = jnp.mean(d * d, axis=-1, keepdims=True)
    inv = lax.rsqrt(var + 1e-5)
    return jnp.maximum(d * inv * g + be, 0.0)


def _pre_body(x_ref, wd_ref, ws_ref, b_ref, p_ref, q_ref):
    x = x_ref[...]
    p_ref[...] = jnp.dot(x, wd_ref[...], preferred_element_type=jnp.float32) + b_ref[...]
    q_ref[...] = jnp.dot(x, ws_ref[...], preferred_element_type=jnp.float32)


def _mlp_body(h_ref, g1_ref, be1_ref, w2_ref, b2_ref, g2_ref, be2_ref, o_ref):
    h = _ln_relu(h_ref[...], g1_ref[...], be1_ref[...])
    h = jnp.dot(h, w2_ref[...], preferred_element_type=jnp.float32) + b2_ref[...]
    o_ref[...] = _ln_relu(h, g2_ref[...], be2_ref[...])


def _upd_body(x_ref, parts_ref, wx_ref, wa_ref, b_ref, g_ref, be_ref, o_ref):
    x = x_ref[...]
    agg = parts_ref[0] + parts_ref[1]
    h = (jnp.dot(x, wx_ref[...], preferred_element_type=jnp.float32)
         + jnp.dot(agg, wa_ref[...], preferred_element_type=jnp.float32)
         + b_ref[...])
    o_ref[...] = x + _ln_relu(h, g_ref[...], be_ref[...])


def _full(shape):
    return pl.BlockSpec(shape, lambda i: (0,) * len(shape))


def _tc_pre(x_pad, wd, ws, b, n_pad, d, bn):
    grid = (n_pad // bn,)
    return pl.pallas_call(
        _pre_body,
        grid=grid,
        in_specs=[pl.BlockSpec((bn, d), lambda i: (i, 0)),
                  _full((d, d)), _full((d, d)), _full((1, d))],
        out_specs=[pl.BlockSpec((bn, d), lambda i: (i, 0)),
                   pl.BlockSpec((bn, d), lambda i: (i, 0))],
        out_shape=[jax.ShapeDtypeStruct((n_pad, d), jnp.float32),
                   jax.ShapeDtypeStruct((n_pad, d), jnp.float32)],
    )(x_pad, wd, ws, b)


def _tc_mlp(msg, g1, be1, w2, b2, g2, be2, e_pad, d, be_blk):
    grid = (e_pad // be_blk,)
    return pl.pallas_call(
        _mlp_body,
        grid=grid,
        in_specs=[pl.BlockSpec((be_blk, d), lambda i: (i, 0)),
                  _full((1, d)), _full((1, d)), _full((d, d)),
                  _full((1, d)), _full((1, d)), _full((1, d))],
        out_specs=pl.BlockSpec((be_blk, d), lambda i: (i, 0)),
        out_shape=jax.ShapeDtypeStruct((e_pad, d), jnp.float32),
    )(msg, g1, be1, w2, b2, g2, be2)


def _tc_update(x, parts, wx, wa, b, g, be, n, d, bn):
    grid = (n // bn,)
    return pl.pallas_call(
        _upd_body,
        grid=grid,
        in_specs=[pl.BlockSpec((bn, d), lambda i: (i, 0)),
                  pl.BlockSpec((2, bn, d), lambda i: (0, i, 0)),
                  _full((d, d)), _full((d, d)),
                  _full((1, d)), _full((1, d)), _full((1, d))],
        out_specs=pl.BlockSpec((bn, d), lambda i: (i, 0)),
        out_shape=jax.ShapeDtypeStruct((n, d), jnp.float32),
    )(x, parts, wx, wa, b, g, be)


def _make_gather(k_chunks, e_pad, n_pad, d):
    mesh = plsc.VectorSubcoreMesh(core_axis_name="c", subcore_axis_name="s")
    assert k_chunks >= 2 and k_chunks % 2 == 1

    @functools.partial(
        pl.kernel,
        out_type=jax.ShapeDtypeStruct((e_pad, d), jnp.float32),
        mesh=mesh,
        scratch_types=[
            pltpu.VMEM((k_chunks, LANE), jnp.int32),
            pltpu.VMEM((k_chunks, LANE), jnp.int32),
            pltpu.VMEM((LANE, d), jnp.float32),
            pltpu.VMEM((LANE, d), jnp.float32),
            pltpu.VMEM((LANE, d), jnp.float32),
            pltpu.VMEM((LANE, d), jnp.float32),
            pltpu.SemaphoreType.DMA, pltpu.SemaphoreType.DMA,
            pltpu.SemaphoreType.DMA, pltpu.SemaphoreType.DMA,
            pltpu.SemaphoreType.DMA, pltpu.SemaphoreType.DMA,
        ],
    )
    def gather_kernel(p_hbm, q_hbm, dst_hbm, src_hbm, msg_hbm,
                      idxd, idxs, bp0, bp1, bq0, bq1,
                      sp0, sp1, sq0, sq1, so0, so1):
        c = lax.axis_index("c")
        s = lax.axis_index("s")
        w = c * NS + s
        row0 = w * k_chunks
        pltpu.sync_copy(dst_hbm.at[w], idxd)
        pltpu.sync_copy(src_hbm.at[w], idxs)
        bp = (bp0, bp1)
        bq = (bq0, bq1)
        sp = (sp0, sp1)
        sq = (sq0, sq1)
        so = (so0, so1)

        def issue_gathers(j, b):
            pltpu.async_copy(p_hbm.at[idxd.at[j]], bp[b], sp[b])
            pltpu.async_copy(q_hbm.at[idxs.at[j]], bq[b], sq[b])

        def wait_gathers(j, b):
            pltpu.make_async_copy(p_hbm.at[idxd.at[j]], bp[b], sp[b]).wait()
            pltpu.make_async_copy(q_hbm.at[idxs.at[j]], bq[b], sq[b]).wait()

        def add_chunk(b):
            def _add(r, carry):
                for rr in range(4):
                    for t in range(d // 16):
                        ix = (r * 4 + rr, pl.ds(t * 16, 16))
                        plsc.addupdate(bp[b].at[ix], bq[b][ix])
                return carry

            lax.fori_loop(0, LANE // 4, _add, 0)

        def store(j, b):
            pltpu.async_copy(bp[b], msg_hbm.at[pl.ds((row0 + j) * LANE, LANE)],
                             so[b])

        def wait_store(j, b):
            pltpu.make_async_copy(
                bp[b], msg_hbm.at[pl.ds((row0 + j) * LANE, LANE)], so[b]).wait()

        issue_gathers(0, 0)
        wait_gathers(0, 0)
        issue_gathers(1, 1)
        add_chunk(0)
        store(0, 0)

        def outer(i, carry):
            for off in (1, 2):
                j = 2 * i + off
                b = off % 2
                wait_gathers(j, b)
                wait_store(j - 1, 1 - b)

                @pl.when(j < k_chunks - 1)
                def _issue():
                    issue_gathers(j + 1, 1 - b)

                add_chunk(b)
                store(j, b)
            return carry

        lax.fori_loop(0, (k_chunks - 1) // 2, outer, 0)
        wait_store(k_chunks - 1, 0)

    return gather_kernel


def _make_scatter(k_chunks, rows2d, n_pad, d):
    mesh = plsc.VectorSubcoreMesh(core_axis_name="c", subcore_axis_name="s")
    zrows = n_pad // NS // LANE

    assert k_chunks >= 2 and k_chunks % 2 == 1

    @functools.partial(
        pl.kernel,
        out_type=jax.ShapeDtypeStruct((NC, n_pad, d), jnp.float32),
        mesh=mesh,
        scratch_types=[
            pltpu.VMEM((k_chunks, LANE), jnp.int32),
            pltpu.VMEM((LANE, d), jnp.float32),
            pltpu.VMEM((LANE, d), jnp.float32),
            pltpu.VMEM_SHARED((n_pad, d), jnp.float32),
            pltpu.SemaphoreType.DMA, pltpu.SemaphoreType.DMA,
            pltpu.SemaphoreType.DMA, pltpu.SemaphoreType.DMA,
        ],
    )
    def scatter_kernel(m_hbm, dst_hbm, out_hbm, idxd, buf0, buf1, agg,
                       sl0, sl1, ss0, ss1):
        c = lax.axis_index("c")
        s = lax.axis_index("s")
        buf = (buf0, buf1)
        sl = (sl0, sl1)
        ss = (ss0, ss1)

        zero16 = jnp.zeros((16,), jnp.float32)

        def zrow(r, carry):
            for t in range(d // 16):
                buf0[r, pl.ds(t * 16, 16)] = zero16
            return carry

        lax.fori_loop(0, LANE, zrow, 0)
        for z in range(zrows):
            pltpu.sync_copy(buf0, agg.at[pl.ds((s * zrows + z) * LANE, LANE)])
        plsc.subcore_barrier()

        w = c * NS + s
        row0 = w * k_chunks
        pltpu.sync_copy(dst_hbm.at[w], idxd)

        def issue_load(j, b):
            pltpu.async_copy(m_hbm.at[pl.ds((row0 + j) * LANE, LANE)], buf[b],
                             sl[b])

        def wait_load(j, b):
            pltpu.make_async_copy(
                m_hbm.at[pl.ds((row0 + j) * LANE, LANE)], buf[b], sl[b]).wait()

        def issue_scat(j, b):
            pltpu.async_copy(buf[b], agg.at[idxd.at[j]], ss[b], add=True)

        def wait_scat(j, b):
            pltpu.make_async_copy(buf[b], agg.at[idxd.at[j]], ss[b]).wait()

        issue_load(0, 0)
        wait_load(0, 0)
        issue_load(1, 1)
        issue_scat(0, 0)

        def chunk(i, carry):
            for off in (1, 2):
                j = 2 * i + off
                b = off % 2
                wait_load(j, b)
                wait_scat(j - 1, 1 - b)

                @pl.when(j < k_chunks - 1)
                def _issue():
                    issue_load(j + 1, 1 - b)

                issue_scat(j, b)
            return carry

        lax.fori_loop(0, (k_chunks - 1) // 2, chunk, 0)
        wait_scat(k_chunks - 1, (k_chunks - 1) % 2)
        plsc.subcore_barrier()

        rpt = n_pad // NS
        pltpu.sync_copy(agg.at[pl.ds(s * rpt, rpt)],
                        out_hbm.at[c, pl.ds(s * rpt, rpt)])

    return scatter_kernel


def kernel(node_features, edge_index, params):
    n, d = node_features.shape
    e = edge_index.shape[1]
    nw = NC * NS

    k_chunks = -(-e // (LANE * nw))
    e_pad = k_chunks * LANE * nw
    rows2d = e_pad // LANE
    n_pad = -(-(n + 1) // (NS * LANE)) * (NS * LANE)

    src = edge_index[0]
    dst = edge_index[1]
    pad_idx = jnp.full((e_pad - e,), n, dtype=jnp.int32)
    src2 = jnp.concatenate([src, pad_idx]).reshape(nw, k_chunks, LANE)
    dst2 = jnp.concatenate([dst, pad_idx]).reshape(nw, k_chunks, LANE)

    gather_k = _make_gather(k_chunks, e_pad, n_pad, d)
    scatter_k = _make_scatter(k_chunks, rows2d, n_pad, d)

    bn_pre = 1024 if n_pad % 1024 == 0 else NS * LANE
    bn_upd = 1000 if n % 1000 == 0 else n
    be_blk = 4096 if e_pad % 4096 == 0 else LANE * nw

    x = node_features
    for bp in params:
        l1, l2 = bp["msg"]
        l3 = bp["upd"][0]
        x_pad = jnp.pad(x, ((0, n_pad - n), (0, 0)))
        p, q = _tc_pre(x_pad, l1["W"][:d], l1["W"][d:], l1["b"][None],
                       n_pad, d, bn_pre)
        msg = gather_k(p, q, dst2, src2)
        m = _tc_mlp(msg, l1["g"][None], l1["be"][None], l2["W"], l2["b"][None],
                    l2["g"][None], l2["be"][None], e_pad, d, be_blk)
        parts = scatter_k(m, dst2)
        x = _tc_update(x, parts, l3["W"][:d], l3["W"][d:], l3["b"][None],
                       l3["g"][None], l3["be"][None], n, d, bn_upd)
    return x


# speedup vs baseline: 1.2410x; 1.1270x over previous
import functools

import jax
import jax.numpy as jnp
from jax import lax
from jax.experimental import pallas as pl
from jax.experimental.pallas import tpu as pltpu
from jax.experimental.pallas import tpu_sc as plsc

NC = 2
NS = 16
LANE = 128


def _ln_relu(h, g, be):
    mu = jnp.mean(h, axis=-1, keepdims=True)
    d = h - mu
    var = jnp.mean(d * d, axis=-1, keepdims=True)
    inv = lax.rsqrt(var + 1e-5)
    return jnp.maximum(d * inv * g + be, 0.0)


def _pre_body(x_ref, wd_ref, ws_ref, b_ref, p_ref, q_ref):
    x = x_ref[...]
    p_ref[...] = jnp.dot(x, wd_ref[...], preferred_element_type=jnp.float32) + b_ref[...]
    q_ref[...] = jnp.dot(x, ws_ref[...], preferred_element_type=jnp.float32)


def _mlp_body(h_ref, g1_ref, be1_ref, w2_ref, b2_ref, g2_ref, be2_ref, o_ref):
    h = _ln_relu(h_ref[...], g1_ref[...], be1_ref[...])
    h = jnp.dot(h, w2_ref[...], preferred_element_type=jnp.float32) + b2_ref[...]
    o_ref[...] = _ln_relu(h, g2_ref[...], be2_ref[...])


def _upd_body(x_ref, parts_ref, wx_ref, wa_ref, b_ref, g_ref, be_ref, o_ref):
    x = x_ref[...]
    agg = parts_ref[0] + parts_ref[1]
    h = (jnp.dot(x, wx_ref[...], preferred_element_type=jnp.float32)
         + jnp.dot(agg, wa_ref[...], preferred_element_type=jnp.float32)
         + b_ref[...])
    o_ref[...] = x + _ln_relu(h, g_ref[...], be_ref[...])


def _full(shape):
    return pl.BlockSpec(shape, lambda i: (0,) * len(shape))


def _tc_pre(x_pad, wd, ws, b, n_pad, d, bn):
    grid = (n_pad // bn,)
    return pl.pallas_call(
        _pre_body,
        grid=grid,
        in_specs=[pl.BlockSpec((bn, d), lambda i: (i, 0)),
                  _full((d, d)), _full((d, d)), _full((1, d))],
        out_specs=[pl.BlockSpec((bn, d), lambda i: (i, 0)),
                   pl.BlockSpec((bn, d), lambda i: (i, 0))],
        out_shape=[jax.ShapeDtypeStruct((n_pad, d), jnp.float32),
                   jax.ShapeDtypeStruct((n_pad, d), jnp.float32)],
    )(x_pad, wd, ws, b)


def _tc_mlp(msg, g1, be1, w2, b2, g2, be2, e_pad, d, be_blk):
    grid = (e_pad // be_blk,)
    return pl.pallas_call(
        _mlp_body,
        grid=grid,
        in_specs=[pl.BlockSpec((be_blk, d), lambda i: (i, 0)),
                  _full((1, d)), _full((1, d)), _full((d, d)),
                  _full((1, d)), _full((1, d)), _full((1, d))],
        out_specs=pl.BlockSpec((be_blk, d), lambda i: (i, 0)),
        out_shape=jax.ShapeDtypeStruct((e_pad, d), jnp.float32),
    )(msg, g1, be1, w2, b2, g2, be2)


def _tc_update(x, parts, wx, wa, b, g, be, n, d, bn):
    grid = (n // bn,)
    return pl.pallas_call(
        _upd_body,
        grid=grid,
        in_specs=[pl.BlockSpec((bn, d), lambda i: (i, 0)),
                  pl.BlockSpec((2, bn, d), lambda i: (0, i, 0)),
                  _full((d, d)), _full((d, d)),
                  _full((1, d)), _full((1, d)), _full((1, d))],
        out_specs=pl.BlockSpec((bn, d), lambda i: (i, 0)),
        out_shape=jax.ShapeDtypeStruct((n, d), jnp.float32),
    )(x, parts, wx, wa, b, g, be)


def _make_gather(k_chunks, e_pad, n_pad, d):
    mesh = plsc.VectorSubcoreMesh(core_axis_name="c", subcore_axis_name="s")
    assert k_chunks >= 3

    @functools.partial(
        pl.kernel,
        out_type=jax.ShapeDtypeStruct((e_pad, d), jnp.float32),
        mesh=mesh,
        scratch_types=[
            pltpu.VMEM((k_chunks, LANE), jnp.int32),
            pltpu.VMEM((k_chunks, LANE), jnp.int32),
            pltpu.VMEM((LANE, d), jnp.float32),
            pltpu.VMEM((LANE, d), jnp.float32),
            pltpu.VMEM((LANE, d), jnp.float32),
            pltpu.VMEM((LANE, d), jnp.float32),
            pltpu.SemaphoreType.DMA, pltpu.SemaphoreType.DMA,
            pltpu.SemaphoreType.DMA, pltpu.SemaphoreType.DMA,
            pltpu.SemaphoreType.DMA, pltpu.SemaphoreType.DMA,
        ],
    )
    def gather_kernel(p_hbm, q_hbm, dst_hbm, src_hbm, msg_hbm,
                      idxd, idxs, bp0, bp1, bq0, bq1,
                      sp0, sp1, sq0, sq1, so0, so1):
        c = lax.axis_index("c")
        s = lax.axis_index("s")
        w = c * NS + s
        row0 = w * k_chunks
        pltpu.sync_copy(dst_hbm.at[w], idxd)
        pltpu.sync_copy(src_hbm.at[w], idxs)
        bp = (bp0, bp1)
        bq = (bq0, bq1)
        sp = (sp0, sp1)
        sq = (sq0, sq1)
        so = (so0, so1)

        def issue_gathers(j, b):
            pltpu.async_copy(p_hbm.at[idxd.at[j]], bp[b], sp[b])
            pltpu.async_copy(q_hbm.at[idxs.at[j]], bq[b], sq[b])

        def wait_gathers(j, b):
            pltpu.make_async_copy(p_hbm.at[idxd.at[j]], bp[b], sp[b]).wait()
            pltpu.make_async_copy(q_hbm.at[idxs.at[j]], bq[b], sq[b]).wait()

        def add_chunk(b):
            def _add(r, carry):
                for rr in range(4):
                    for t in range(d // 16):
                        ix = (r * 4 + rr, pl.ds(t * 16, 16))
                        plsc.addupdate(bp[b].at[ix], bq[b][ix])
                return carry

            lax.fori_loop(0, LANE // 4, _add, 0)

        def store(j, b):
            pltpu.async_copy(bp[b], msg_hbm.at[pl.ds((row0 + j) * LANE, LANE)],
                             so[b])

        def wait_store(j, b):
            pltpu.make_async_copy(
                bp[b], msg_hbm.at[pl.ds((row0 + j) * LANE, LANE)], so[b]).wait()

        issue_gathers(0, 0)
        wait_gathers(0, 0)
        issue_gathers(1, 1)
        add_chunk(0)
        store(0, 0)

        def outer(i, carry):
            for off in (1, 2):
                j = 2 * i + off
                b = off % 2
                wait_gathers(j, b)
                wait_store(j - 1, 1 - b)

                @pl.when(j < k_chunks - 1)
                def _issue():
                    issue_gathers(j + 1, 1 - b)

                add_chunk(b)
                store(j, b)
            return carry

        lax.fori_loop(0, (k_chunks - 1) // 2, outer, 0)
        if k_chunks % 2 == 0:
            j = k_chunks - 1
            wait_gathers(j, 1)
            wait_store(j - 1, 0)
            add_chunk(1)
            store(j, 1)
            wait_store(j, 1)
        else:
            wait_store(k_chunks - 1, 0)

    return gather_kernel


def _make_scatter(k_chunks, n_pad, d, accumulate):
    mesh = plsc.VectorSubcoreMesh(core_axis_name="c", subcore_axis_name="s")
    zrows = n_pad // NS // LANE
    rpt = n_pad // NS
    assert k_chunks >= 3

    scratch = [
        pltpu.VMEM((k_chunks, LANE), jnp.int32),
        pltpu.VMEM((LANE, d), jnp.float32),
        pltpu.VMEM((LANE, d), jnp.float32),
        pltpu.VMEM_SHARED((n_pad, d), jnp.float32),
        pltpu.SemaphoreType.DMA, pltpu.SemaphoreType.DMA,
        pltpu.SemaphoreType.DMA, pltpu.SemaphoreType.DMA,
    ]

    def body(m_hbm, dst_hbm, prev_hbm, out_hbm, idxd, buf0, buf1, agg,
             sl0, sl1, ss0, ss1):
        c = lax.axis_index("c")
        s = lax.axis_index("s")
        buf = (buf0, buf1)
        sl = (sl0, sl1)
        ss = (ss0, ss1)
        rows_t = pl.ds(s * rpt, rpt)

        if accumulate:
            pltpu.sync_copy(prev_hbm.at[c, rows_t], agg.at[rows_t])
        else:
            zero16 = jnp.zeros((16,), jnp.float32)

            def zrow(r, carry):
                for t in range(d // 16):
                    buf0[r, pl.ds(t * 16, 16)] = zero16
                return carry

            lax.fori_loop(0, LANE, zrow, 0)
            for z in range(zrows):
                pltpu.sync_copy(buf0, agg.at[pl.ds((s * zrows + z) * LANE, LANE)])
        plsc.subcore_barrier()

        w = c * NS + s
        row0 = w * k_chunks
        pltpu.sync_copy(dst_hbm.at[w], idxd)

        def issue_load(j, b):
            pltpu.async_copy(m_hbm.at[pl.ds((row0 + j) * LANE, LANE)], buf[b],
                             sl[b])

        def wait_load(j, b):
            pltpu.make_async_copy(
                m_hbm.at[pl.ds((row0 + j) * LANE, LANE)], buf[b], sl[b]).wait()

        def issue_scat(j, b):
            pltpu.async_copy(buf[b], agg.at[idxd.at[j]], ss[b], add=True)

        def wait_scat(j, b):
            pltpu.make_async_copy(buf[b], agg.at[idxd.at[j]], ss[b]).wait()

        issue_load(0, 0)
        wait_load(0, 0)
        issue_load(1, 1)
        issue_scat(0, 0)

        def chunk(i, carry):
            for off in (1, 2):
                j = 2 * i + off
                b = off % 2
                wait_load(j, b)
                wait_scat(j - 1, 1 - b)

                @pl.when(j < k_chunks - 1)
                def _issue():
                    issue_load(j + 1, 1 - b)

                issue_scat(j, b)
            return carry

        lax.fori_loop(0, (k_chunks - 1) // 2, chunk, 0)
        if k_chunks % 2 == 0:
            j = k_chunks - 1
            wait_load(j, 1)
            wait_scat(j - 1, 0)
            issue_scat(j, 1)
        wait_scat(k_chunks - 1, (k_chunks - 1) % 2)
        plsc.subcore_barrier()

        pltpu.sync_copy(agg.at[rows_t], out_hbm.at[c, rows_t])

    if accumulate:
        kernel_fn = body
    else:
        def kernel_fn(m_hbm, dst_hbm, out_hbm, *rest):
            body(m_hbm, dst_hbm, None, out_hbm, *rest)

    return functools.partial(
        pl.kernel,
        out_type=jax.ShapeDtypeStruct((NC, n_pad, d), jnp.float32),
        mesh=mesh,
        scratch_types=scratch,
    )(kernel_fn)


def kernel(node_features, edge_index, params):
    n, d = node_features.shape
    e = edge_index.shape[1]
    nw = NC * NS

    k_total = -(-e // (LANE * nw))
    e_pad = k_total * LANE * nw
    rows2d = e_pad // LANE
    n_pad = -(-(n + 1) // (NS * LANE)) * (NS * LANE)

    k_h0 = (k_total + 1) // 2
    k_h1 = k_total - k_h0
    e_h0 = k_h0 * LANE * nw
    e_h1 = k_h1 * LANE * nw

    src = edge_index[0]
    dst = edge_index[1]
    pad_idx = jnp.full((e_pad - e,), n, dtype=jnp.int32)
    flat_src = jnp.concatenate([src, pad_idx])
    flat_dst = jnp.concatenate([dst, pad_idx])
    src_h0 = flat_src[:e_h0].reshape(nw, k_h0, LANE)
    dst_h0 = flat_dst[:e_h0].reshape(nw, k_h0, LANE)
    src_h1 = flat_src[e_h0:].reshape(nw, k_h1, LANE)
    dst_h1 = flat_dst[e_h0:].reshape(nw, k_h1, LANE)

    gather0 = _make_gather(k_h0, e_h0, n_pad, d)
    gather1 = _make_gather(k_h1, e_h1, n_pad, d)
    scat0 = _make_scatter(k_h0, n_pad, d, accumulate=False)
    scat1 = _make_scatter(k_h1, n_pad, d, accumulate=True)

    bn_pre = 1024 if n_pad % 1024 == 0 else NS * LANE
    bn_upd = 1000 if n % 1000 == 0 else n
    be0 = 4096 if e_h0 % 4096 == 0 else LANE * nw
    be1 = 4096 if e_h1 % 4096 == 0 else LANE * nw

    x = node_features
    for bp in params:
        l1, l2 = bp["msg"]
        l3 = bp["upd"][0]
        x_pad = jnp.pad(x, ((0, n_pad - n), (0, 0)))
        p, q = _tc_pre(x_pad, l1["W"][:d], l1["W"][d:], l1["b"][None],
                       n_pad, d, bn_pre)
        msg0 = gather0(p, q, dst_h0, src_h0)
        msg1 = gather1(p, q, dst_h1, src_h1)
        m0 = _tc_mlp(msg0, l1["g"][None], l1["be"][None], l2["W"], l2["b"][None],
                     l2["g"][None], l2["be"][None], e_h0, d, be0)
        m1 = _tc_mlp(msg1, l1["g"][None], l1["be"][None], l2["W"], l2["b"][None],
                     l2["g"][None], l2["be"][None], e_h1, d, be1)
        parts0 = scat0(m0, dst_h0)
        parts = scat1(m1, dst_h1, parts0)
        x = _tc_update(x, parts, l3["W"][:d], l3["W"][d:], l3["b"][None],
                       l3["g"][None], l3["be"][None], n, d, bn_upd)
    return x
